# Initial kernel scaffold; baseline (speedup 1.0000x reference)
import jax
import jax.numpy as jnp
from jax.experimental import pallas as pl


def kernel(x, edge_index, W1, b1, W2, b2, W3, b3):
    raise NotImplementedError("write your pallas kernel here")


# speedup vs baseline: 4.2728x; 4.2728x over previous
import functools

import jax
import jax.numpy as jnp
from jax import lax
from jax.experimental import pallas as pl
from jax.experimental.pallas import tpu as pltpu
from jax.experimental.pallas import tpu_sc as plsc

N = 10000
NP = 10240
E = 320000
D = 128
NC = 2
NS = 16
NW = NC * NS
EPW = E // NW
EPWP = 10240
C = 128
CAPC = EPWP // C
NR = 5
RANGE = NP // NR
TRASH = RANGE
PADDST = 10200
AROWS = 2080
TPS = AROWS // NS
WPS = RANGE // NS


def _zero_vmem(ref, rows, width):
    z16 = jnp.zeros((16,), jnp.float32)

    def row(i, _):
        for j in range(width // 16):
            ref[i, pl.ds(j * 16, 16)] = z16
        return 0

    lax.fori_loop(0, rows, row, 0)


_MESH = plsc.VectorSubcoreMesh(core_axis_name="c", subcore_axis_name="s")


@functools.partial(
    pl.kernel,
    out_type=jax.ShapeDtypeStruct((NC, NP, 16), jnp.float32),
    mesh=_MESH,
    scratch_types=[
        pltpu.VMEM((C,), jnp.int32),
        pltpu.VMEM((C, 16), jnp.float32),
        pltpu.VMEM((TPS, 16), jnp.float32),
        pltpu.VMEM_SHARED((NP, 16), jnp.float32),
    ],
)
def _deg_kernel(dst_hbm, out_hbm, dst_v, ones_v, zbuf, acc):
    c = lax.axis_index("c")
    s = lax.axis_index("s")
    wid = s * NC + c
    rps = NP // NS

    _zero_vmem(zbuf, TPS, 16)
    for k in range(rps // TPS + 1):
        base = k * TPS
        if base + TPS > rps:
            base = rps - TPS
        pltpu.sync_copy(zbuf, acc.at[pl.ds(s * rps + base, TPS)])

    onehot = jnp.where(lax.iota(jnp.int32, 16) == 0, 1.0, 0.0).astype(
        jnp.float32)

    def mk(i, _):
        ones_v[i, pl.ds(0, 16)] = onehot
        return 0

    lax.fori_loop(0, C, mk, 0)
    plsc.subcore_barrier()

    def chunk(j, _):
        pltpu.sync_copy(dst_hbm.at[wid, pl.ds(j * C, C)], dst_v)
        pltpu.sync_copy(ones_v, acc.at[dst_v], add=True)
        return 0

    lax.fori_loop(0, CAPC, chunk, 0)
    plsc.subcore_barrier()

    pltpu.sync_copy(acc.at[pl.ds(s * rps, rps)], out_hbm.at[c, pl.ds(s * rps, rps)])


@functools.partial(
    pl.kernel,
    out_type=[
        jax.ShapeDtypeStruct((NR, NW, EPWP), jnp.int32),
        jax.ShapeDtypeStruct((NR, NW, EPWP), jnp.int32),
        jax.ShapeDtypeStruct((NW, 16), jnp.int32),
    ],
    mesh=_MESH,
    compiler_params=pltpu.CompilerParams(needs_layout_passes=False),
    scratch_types=[
        pltpu.VMEM((EPWP,), jnp.int32),
        pltpu.VMEM((EPWP,), jnp.int32),
        pltpu.VMEM((EPWP + 16,), jnp.int32),
        pltpu.VMEM((EPWP + 16,), jnp.int32),
        pltpu.VMEM((16,), jnp.int32),
    ],
)
def _part_kernel(src_hbm, dst_hbm, psrc_hbm, pdst_hbm, cnt_hbm,
                 sfv, dfv, osrc, odst, cntv):
    c = lax.axis_index("c")
    s = lax.axis_index("s")
    wid = s * NC + c

    pltpu.sync_copy(src_hbm.at[wid], sfv)
    pltpu.sync_copy(dst_hbm.at[wid], dfv)

    lanes = lax.iota(jnp.int32, 16)
    zero16 = jnp.zeros((16,), jnp.int32)
    trash16 = jnp.full((16,), TRASH, jnp.int32)
    cvec = zero16

    for r in range(NR):
        lo = r * RANGE

        def prefill(i, _):
            osrc[pl.ds(i * 16, 16)] = zero16
            odst[pl.ds(i * 16, 16)] = trash16
            return 0

        lax.fori_loop(0, (EPWP + 16) // 16, prefill, 0)

        def grp(g, off):
            sv = sfv[pl.ds(g * 16, 16)]
            dv = dfv[pl.ds(g * 16, 16)]
            m = (dv >= lo) & (dv < lo + RANGE)
            plsc.store_compressed(osrc.at[pl.ds(off, 16)], sv, mask=m)
            plsc.store_compressed(odst.at[pl.ds(off, 16)], dv - lo, mask=m)
            return off + plsc.all_reduce_population_count(m)[0]

        cnt = lax.fori_loop(0, EPWP // 16, grp, 0)
        nch = (cnt + (C - 1)) // C
        cvec = jnp.where(lanes == r, nch, cvec)
        pltpu.sync_copy(osrc.at[pl.ds(0, EPWP)], psrc_hbm.at[r, wid])
        pltpu.sync_copy(odst.at[pl.ds(0, EPWP)], pdst_hbm.at[r, wid])

    cntv[...] = cvec
    pltpu.sync_copy(cntv, cnt_hbm.at[wid])


@functools.partial(
    pl.kernel,
    out_type=jax.ShapeDtypeStruct((NC, NP, D), jnp.float32),
    mesh=_MESH,
    scratch_types=[
        pltpu.VMEM((C,), jnp.int32),
        pltpu.VMEM((C,), jnp.int32),
        pltpu.VMEM((C, D), jnp.float32),
        pltpu.VMEM((TPS, D), jnp.float32),
        pltpu.VMEM((16,), jnp.int32),
        pltpu.VMEM_SHARED((AROWS, D), jnp.float32),
        pltpu.SemaphoreType.DMA,
    ],
)
def _prop_kernel(y_hbm, psrc_hbm, pdst_hbm, cnt_hbm, out_hbm,
                 srcl, dstl, rows_v, zbuf, cntv, acc, sem):
    c = lax.axis_index("c")
    s = lax.axis_index("s")
    wid = s * NC + c

    pltpu.sync_copy(cnt_hbm.at[wid], cntv)
    cv = cntv[...]
    _zero_vmem(zbuf, TPS, D)

    for r in range(NR):
        pltpu.sync_copy(zbuf, acc.at[pl.ds(s * TPS, TPS)])
        plsc.subcore_barrier()

        def chunk(q, _):
            pltpu.sync_copy(psrc_hbm.at[r, wid, pl.ds(q * C, C)], srcl)
            pltpu.sync_copy(pdst_hbm.at[r, wid, pl.ds(q * C, C)], dstl)
            pltpu.async_copy(y_hbm.at[srcl], rows_v, sem).wait()
            pltpu.sync_copy(rows_v, acc.at[dstl], add=True)
            return 0

        lax.fori_loop(0, cv[r], chunk, 0)
        plsc.subcore_barrier()

        pltpu.sync_copy(acc.at[pl.ds(s * WPS, WPS)],
                        out_hbm.at[c, pl.ds(r * RANGE + s * WPS, WPS)])
        plsc.subcore_barrier()


R = 2048


def _pre_body(dacc_ref, x_ref, w_ref, dinv_ref, y_ref):
    deg = 1.0 + dacc_ref[0, :, 0] + dacc_ref[1, :, 0]
    dv = lax.rsqrt(deg)[:, None]
    dinv_ref[...] = dv
    y_ref[...] = jnp.dot(x_ref[...] * dv, w_ref[...],
                         preferred_element_type=jnp.float32)


def _tc_pre(dacc, x, w1):
    return pl.pallas_call(
        _pre_body,
        grid=(NP // R,),
        in_specs=[
            pl.BlockSpec((NC, R, 16), lambda i: (0, i, 0)),
            pl.BlockSpec((R, D), lambda i: (i, 0)),
            pl.BlockSpec((D, D), lambda i: (0, 0)),
        ],
        out_specs=[
            pl.BlockSpec((R, 1), lambda i: (i, 0)),
            pl.BlockSpec((R, D), lambda i: (i, 0)),
        ],
        out_shape=[
            jax.ShapeDtypeStruct((NP, 1), jnp.float32),
            jax.ShapeDtypeStruct((NP, D), jnp.float32),
        ],
    )(dacc, x, w1)


def _mid_body(y_ref, s_ref, dinv_ref, b_ref, w_ref, yn_ref):
    dv = dinv_ref[...]
    out = dv * (y_ref[...] + s_ref[0] + s_ref[1]) + b_ref[...]
    h = jnp.maximum(out, 0.0)
    yn_ref[...] = jnp.dot(h * dv, w_ref[...],
                          preferred_element_type=jnp.float32)


def _tc_mid(y, sacc, dinv, b, wn):
    return pl.pallas_call(
        _mid_body,
        grid=(NP // R,),
        in_specs=[
            pl.BlockSpec((R, D), lambda i: (i, 0)),
            pl.BlockSpec((NC, R, D), lambda i: (0, i, 0)),
            pl.BlockSpec((R, 1), lambda i: (i, 0)),
            pl.BlockSpec((D,), lambda i: (0,)),
            pl.BlockSpec((D, D), lambda i: (0, 0)),
        ],
        out_specs=pl.BlockSpec((R, D), lambda i: (i, 0)),
        out_shape=jax.ShapeDtypeStruct((NP, D), jnp.float32),
    )(y, sacc, dinv, b, wn)


def _fin_body(y_ref, s_ref, dinv_ref, b_ref, o_ref):
    o_ref[...] = (dinv_ref[...] * (y_ref[...] + s_ref[0] + s_ref[1])
                  + b_ref[...])


def _tc_fin(y, sacc, dinv, b):
    return pl.pallas_call(
        _fin_body,
        grid=(NP // R,),
        in_specs=[
            pl.BlockSpec((R, D), lambda i: (i, 0)),
            pl.BlockSpec((NC, R, D), lambda i: (0, i, 0)),
            pl.BlockSpec((R, 1), lambda i: (i, 0)),
            pl.BlockSpec((D,), lambda i: (0,)),
        ],
        out_specs=pl.BlockSpec((R, D), lambda i: (i, 0)),
        out_shape=jax.ShapeDtypeStruct((NP, D), jnp.float32),
    )(y, sacc, dinv, b)


def kernel(x, edge_index, W1, b1, W2, b2, W3, b3):
    pad = ((0, 0), (0, EPWP - EPW))
    src_f = jnp.pad(edge_index[0].reshape(NW, EPW), pad)
    dst_f = jnp.pad(edge_index[1].reshape(NW, EPW), pad,
                    constant_values=PADDST)
    xp = jnp.pad(x, ((0, NP - N), (0, 0)))

    psrc, pdst, cnts = _part_kernel(src_f, dst_f)
    dacc = _deg_kernel(dst_f)
    dinv, y = _tc_pre(dacc, xp, W1)
    s = _prop_kernel(y, psrc, pdst, cnts)
    y = _tc_mid(y, s, dinv, b1, W2)
    s = _prop_kernel(y, psrc, pdst, cnts)
    y = _tc_mid(y, s, dinv, b2, W3)
    s = _prop_kernel(y, psrc, pdst, cnts)
    return _tc_fin(y, s, dinv, b3)[:N]
